# Initial kernel scaffold; baseline (speedup 1.0000x reference)
import jax
import jax.numpy as jnp
from jax.experimental import pallas as pl


def kernel(energy, Z, frame_ids):
    raise NotImplementedError("write your pallas kernel here")


# speedup vs baseline: 62.2491x; 62.2491x over previous
import functools

import jax
import jax.numpy as jnp
from jax import lax
from jax.experimental import pallas as pl
from jax.experimental.pallas import tpu as pltpu
from jax.experimental.pallas import tpu_sc as plsc

N_ATOMS = 262144
N_FRAMES = 4096
N_ELEMS = 94
E_PAD = 128
N_BINS = N_FRAMES * E_PAD
NC = 2
NS = 16
CHUNK = N_ATOMS // (NC * NS)
ROWS = CHUNK // E_PAD
HIST_SLICE = N_BINS // NS
NS_ITERS = 20

_mesh = plsc.VectorSubcoreMesh(core_axis_name="c", subcore_axis_name="s")


def _sc_hist_body(z_hbm, f_hbm, out_hbm, zbuf, fbuf, idx2d, fill, ones_row, hist):
    cid = lax.axis_index("c")
    sid = lax.axis_index("s")
    base = cid * (N_ATOMS // NC) + sid * CHUNK

    pltpu.sync_copy(z_hbm.at[pl.ds(base, CHUNK)], zbuf)
    pltpu.sync_copy(f_hbm.at[pl.ds(base, CHUNK)], fbuf)

    def _zfill(i, _):
        fill[pl.ds(i * 16, 16)] = jnp.zeros((16,), jnp.float32)
        return 0

    lax.fori_loop(0, CHUNK // 16, _zfill, 0)
    for k in range(HIST_SLICE // CHUNK):
        pltpu.sync_copy(fill, hist.at[pl.ds(sid * HIST_SLICE + k * CHUNK, CHUNK)])

    def _ifill(j, _):
        for k in range(E_PAD // 16):
            off = j * E_PAD + k * 16
            z = zbuf[pl.ds(off, 16)]
            f = fbuf[pl.ds(off, 16)]
            idx2d[j, pl.ds(k * 16, 16)] = f * E_PAD + z - 1
        return 0

    lax.fori_loop(0, ROWS, _ifill, 0)

    for k in range(E_PAD // 16):
        ones_row[pl.ds(k * 16, 16)] = jnp.full((16,), 1.0, jnp.float32)

    plsc.subcore_barrier()

    def _scat(j, _):
        pltpu.sync_copy(ones_row, hist.at[idx2d.at[j]], add=True)
        return 0

    lax.fori_loop(0, ROWS, _scat, 0)

    plsc.subcore_barrier()

    pltpu.sync_copy(
        hist.at[pl.ds(sid * HIST_SLICE, HIST_SLICE)],
        out_hbm.at[cid, pl.ds(sid * HIST_SLICE, HIST_SLICE)],
    )


_sc_hist = functools.partial(
    pl.kernel,
    out_type=jax.ShapeDtypeStruct((NC, N_BINS), jnp.float32),
    mesh=_mesh,
    scratch_types=[
        pltpu.VMEM((CHUNK,), jnp.int32),
        pltpu.VMEM((CHUNK,), jnp.int32),
        pltpu.VMEM((ROWS, E_PAD), jnp.int32),
        pltpu.VMEM((CHUNK,), jnp.float32),
        pltpu.VMEM((E_PAD,), jnp.float32),
        pltpu.VMEM_SHARED((N_BINS,), jnp.float32),
    ],
)(_sc_hist_body)


def _mm(a, b):
    return lax.dot_general(a, b, (((1,), (0,)), ((), ())),
                           preferred_element_type=jnp.float32)


def _fit_body(parts_ref, y_ref, out_ref):
    x = parts_ref[0] + parts_ref[1]
    y = y_ref[...]
    G = lax.dot_general(x, x, (((0,), (0,)), ((), ())),
                        preferred_element_type=jnp.float32)
    b = lax.dot_general(x, y, (((0,), (0,)), ((), ())),
                        preferred_element_type=jnp.float32)
    s = jnp.max(jnp.sum(jnp.abs(G), axis=1))
    X0 = G * (1.0 / (s * s))

    def _ns(i, X):
        return 2.0 * X - _mm(X, _mm(G, X))

    X = lax.fori_loop(0, NS_ITERS, _ns, X0)
    beta = _mm(X, b)
    out_ref[...] = y - _mm(x, beta)


_tc_fit = pl.pallas_call(
    _fit_body,
    out_shape=jax.ShapeDtypeStruct((N_FRAMES, 1), jnp.float32),
    in_specs=[
        pl.BlockSpec(memory_space=pltpu.VMEM),
        pl.BlockSpec(memory_space=pltpu.VMEM),
    ],
    out_specs=pl.BlockSpec(memory_space=pltpu.VMEM),
)


def kernel(energy, Z, frame_ids):
    parts = _sc_hist(Z, frame_ids)
    parts3 = parts.reshape(NC, N_FRAMES, E_PAD)
    out = _tc_fit(parts3, energy.reshape(N_FRAMES, 1))
    return out.reshape(N_FRAMES)


# speedup vs baseline: 69.6863x; 1.1195x over previous
import functools

import jax
import jax.numpy as jnp
from jax import lax
from jax.experimental import pallas as pl
from jax.experimental.pallas import tpu as pltpu
from jax.experimental.pallas import tpu_sc as plsc

N_ATOMS = 262144
N_FRAMES = 4096
N_ELEMS = 94
E_PAD = 128
N_BINS = N_FRAMES * E_PAD
NC = 2
NS = 16
CHUNK = N_ATOMS // (NC * NS)
ROWS = CHUNK // E_PAD
HIST_SLICE = N_BINS // NS
NS_ITERS = 18
DEPTH = 16

_mesh = plsc.VectorSubcoreMesh(core_axis_name="c", subcore_axis_name="s")


def _sc_hist_body(z_hbm, f_hbm, out_hbm, zbuf, fbuf, idx2d, fill, ones_row, hist,
                  zsem, fsem, ssem):
    cid = lax.axis_index("c")
    sid = lax.axis_index("s")
    base = cid * (N_ATOMS // NC) + sid * CHUNK

    zcp = pltpu.async_copy(z_hbm.at[pl.ds(base, CHUNK)], zbuf, zsem)
    fcp = pltpu.async_copy(f_hbm.at[pl.ds(base, CHUNK)], fbuf, fsem)

    def _zfill(i, _):
        fill[pl.ds(i * 16, 16)] = jnp.zeros((16,), jnp.float32)
        return 0

    lax.fori_loop(0, CHUNK // 16, _zfill, 0)
    for k in range(HIST_SLICE // CHUNK):
        pltpu.sync_copy(fill, hist.at[pl.ds(sid * HIST_SLICE + k * CHUNK, CHUNK)])

    for k in range(E_PAD // 16):
        ones_row[pl.ds(k * 16, 16)] = jnp.full((16,), 1.0, jnp.float32)

    zcp.wait()
    fcp.wait()

    def _ifill(j, _):
        for k in range(E_PAD // 16):
            off = j * E_PAD + k * 16
            z = zbuf[pl.ds(off, 16)]
            f = fbuf[pl.ds(off, 16)]
            idx2d[j, pl.ds(k * 16, 16)] = f * E_PAD + z - 1
        return 0

    lax.fori_loop(0, ROWS, _ifill, 0)

    plsc.subcore_barrier()

    def _scat(j, _):
        pltpu.async_copy(ones_row, hist.at[idx2d.at[j]], ssem, add=True)

        @pl.when(j >= DEPTH)
        def _drain():
            pltpu.make_async_copy(ones_row, hist.at[idx2d.at[j - DEPTH]], ssem).wait()

        return 0

    lax.fori_loop(0, ROWS, _scat, 0)

    def _tail(j, _):
        pltpu.make_async_copy(
            ones_row, hist.at[idx2d.at[ROWS - DEPTH + j]], ssem).wait()
        return 0

    lax.fori_loop(0, DEPTH, _tail, 0)

    plsc.subcore_barrier()

    pltpu.sync_copy(
        hist.at[pl.ds(sid * HIST_SLICE, HIST_SLICE)],
        out_hbm.at[cid, pl.ds(sid * HIST_SLICE, HIST_SLICE)],
    )


_sc_hist = functools.partial(
    pl.kernel,
    out_type=jax.ShapeDtypeStruct((NC, N_BINS), jnp.float32),
    mesh=_mesh,
    scratch_types=[
        pltpu.VMEM((CHUNK,), jnp.int32),
        pltpu.VMEM((CHUNK,), jnp.int32),
        pltpu.VMEM((ROWS, E_PAD), jnp.int32),
        pltpu.VMEM((CHUNK,), jnp.float32),
        pltpu.VMEM((E_PAD,), jnp.float32),
        pltpu.VMEM_SHARED((N_BINS,), jnp.float32),
        pltpu.SemaphoreType.DMA,
        pltpu.SemaphoreType.DMA,
        pltpu.SemaphoreType.DMA,
    ],
)(_sc_hist_body)


def _mm(a, b):
    return lax.dot_general(a, b, (((1,), (0,)), ((), ())),
                           preferred_element_type=jnp.float32)


def _fit_body(parts_ref, y_ref, out_ref):
    x = parts_ref[0] + parts_ref[1]
    y = y_ref[...]
    G = lax.dot_general(x, x, (((0,), (0,)), ((), ())),
                        preferred_element_type=jnp.float32)
    b = lax.dot_general(x, y, (((0,), (0,)), ((), ())),
                        preferred_element_type=jnp.float32)
    s = jnp.max(jnp.sum(jnp.abs(G), axis=1))
    X0 = G * (1.0 / (s * s))

    def _ns(i, X):
        return 2.0 * X - _mm(X, _mm(G, X))

    X = lax.fori_loop(0, NS_ITERS, _ns, X0)
    beta = _mm(X, b)
    out_ref[...] = y - _mm(x, beta)


_tc_fit = pl.pallas_call(
    _fit_body,
    out_shape=jax.ShapeDtypeStruct((N_FRAMES, 1), jnp.float32),
    in_specs=[
        pl.BlockSpec(memory_space=pltpu.VMEM),
        pl.BlockSpec(memory_space=pltpu.VMEM),
    ],
    out_specs=pl.BlockSpec(memory_space=pltpu.VMEM),
)


def kernel(energy, Z, frame_ids):
    parts = _sc_hist(Z, frame_ids)
    parts3 = parts.reshape(NC, N_FRAMES, E_PAD)
    out = _tc_fit(parts3, energy.reshape(N_FRAMES, 1))
    return out.reshape(N_FRAMES)


# speedup vs baseline: 83.5067x; 1.1983x over previous
import functools

import jax
import jax.numpy as jnp
from jax import lax
from jax.experimental import pallas as pl
from jax.experimental.pallas import tpu as pltpu
from jax.experimental.pallas import tpu_sc as plsc

N_ATOMS = 262144
N_FRAMES = 4096
N_ELEMS = 94
E_PAD = 128
N_BINS = N_FRAMES * E_PAD
NC = 2
NS = 16
CHUNK = N_ATOMS // (NC * NS)
ROWS = CHUNK // E_PAD
HIST_SLICE = N_BINS // NS
NS_ITERS = 18
DEPTH = 16

_mesh = plsc.VectorSubcoreMesh(core_axis_name="c", subcore_axis_name="s")


def _sc_hist_body(z_hbm, f_hbm, out_hbm, zbuf, fbuf, idx2d, fill, ones_row, hist,
                  zsem, fsem, ssem):
    cid = lax.axis_index("c")
    sid = lax.axis_index("s")
    base = cid * (N_ATOMS // NC) + sid * CHUNK

    zcp = pltpu.async_copy(z_hbm.at[pl.ds(base, CHUNK)], zbuf, zsem)
    fcp = pltpu.async_copy(f_hbm.at[pl.ds(base, CHUNK)], fbuf, fsem)

    def _zfill(i, _):
        fill[pl.ds(i * 16, 16)] = jnp.zeros((16,), jnp.float32)
        return 0

    lax.fori_loop(0, CHUNK // 16, _zfill, 0)
    hist_flat = hist
    for k in range(HIST_SLICE // CHUNK):
        pltpu.sync_copy(fill, hist_flat.at[pl.ds(sid * HIST_SLICE + k * CHUNK, CHUNK)])

    for k in range(E_PAD // 16):
        ones_row[pl.ds(k * 16, 16)] = jnp.full((16,), 1.0, jnp.float32)

    zcp.wait()
    fcp.wait()

    def _ifill(j, _):
        for k in range(E_PAD // 16):
            off = j * E_PAD + k * 16
            z = zbuf[pl.ds(off, 16)]
            f = fbuf[pl.ds(off, 16)]
            idx2d[j, pl.ds(k * 16, 16)] = f * E_PAD + z - 1
        return 0

    lax.fori_loop(0, ROWS, _ifill, 0)

    plsc.subcore_barrier()

    def _scat(j, _):
        pltpu.async_copy(ones_row, hist_flat.at[idx2d.at[j]], ssem, add=True)

        @pl.when(j >= DEPTH)
        def _drain():
            pltpu.make_async_copy(ones_row, hist_flat.at[idx2d.at[j - DEPTH]], ssem).wait()

        return 0

    lax.fori_loop(0, ROWS, _scat, 0)

    def _tail(j, _):
        pltpu.make_async_copy(
            ones_row, hist_flat.at[idx2d.at[ROWS - DEPTH + j]], ssem).wait()
        return 0

    lax.fori_loop(0, DEPTH, _tail, 0)

    plsc.subcore_barrier()

    pltpu.sync_copy(
        hist.at[pl.ds(sid * HIST_SLICE, HIST_SLICE)],
        out_hbm.at[pl.ds(cid * N_BINS + sid * HIST_SLICE, HIST_SLICE)],
    )


_sc_hist = functools.partial(
    pl.kernel,
    out_type=jax.ShapeDtypeStruct((NC * N_BINS,), jnp.float32),
    mesh=_mesh,
    scratch_types=[
        pltpu.VMEM((CHUNK,), jnp.int32),
        pltpu.VMEM((CHUNK,), jnp.int32),
        pltpu.VMEM((ROWS, E_PAD), jnp.int32),
        pltpu.VMEM((CHUNK,), jnp.float32),
        pltpu.VMEM((E_PAD,), jnp.float32),
        pltpu.VMEM_SHARED((N_BINS,), jnp.float32),
        pltpu.SemaphoreType.DMA,
        pltpu.SemaphoreType.DMA,
        pltpu.SemaphoreType.DMA,
    ],
)(_sc_hist_body)


def _mm(a, b):
    return lax.dot_general(a, b, (((1,), (0,)), ((), ())),
                           preferred_element_type=jnp.float32)


def _fit_body(parts_ref, y_ref, out_ref):
    x = parts_ref[:N_FRAMES] + parts_ref[N_FRAMES:]
    y = y_ref[...]
    G = lax.dot_general(x, x, (((0,), (0,)), ((), ())),
                        preferred_element_type=jnp.float32)
    b = lax.dot_general(x, y, (((0,), (0,)), ((), ())),
                        preferred_element_type=jnp.float32)
    s = jnp.max(jnp.sum(jnp.abs(G), axis=1))
    X0 = G * (1.0 / (s * s))

    def _ns(i, X):
        return 2.0 * X - _mm(X, _mm(G, X))

    X = lax.fori_loop(0, NS_ITERS, _ns, X0)
    beta = _mm(X, b)
    res = y - _mm(x, beta)
    out_ref[...] = res.reshape(N_FRAMES // E_PAD, E_PAD)


_tc_fit = pl.pallas_call(
    _fit_body,
    out_shape=jax.ShapeDtypeStruct((N_FRAMES // E_PAD, E_PAD), jnp.float32),
    in_specs=[
        pl.BlockSpec(memory_space=pltpu.VMEM),
        pl.BlockSpec(memory_space=pltpu.VMEM),
    ],
    out_specs=pl.BlockSpec(memory_space=pltpu.VMEM),
)


def kernel(energy, Z, frame_ids):
    parts = _sc_hist(Z, frame_ids)
    parts2d = parts.reshape(NC * N_FRAMES, E_PAD)
    out = _tc_fit(parts2d, energy.reshape(N_FRAMES, 1))
    return out.reshape(N_FRAMES)
